# Initial kernel scaffold; baseline (speedup 1.0000x reference)
import jax
import jax.numpy as jnp
from jax.experimental import pallas as pl


def kernel(X, mask, wh_n, wsW_n, wsb_n, wv_n, wh_e, wsW_e, wsb_e, wv_e, gamma_n, beta_n, gamma_e, beta_e):
    raise NotImplementedError("write your pallas kernel here")


# speedup vs baseline: 2.3651x; 2.3651x over previous
import functools
import numpy as np
import jax
import jax.numpy as jnp
from jax.experimental import pallas as pl

TOP_K = 30
KPAD = 32
NUM_RBF = 16
NUM_POS = 16
BIG = 1e30


def _knn_body(xr_x, xr_y, xr_z, xc_x, xc_y, xc_z,
              oidx, od, oxx, oxy, oxz):
    rx = xr_x[0]
    ry = xr_y[0]
    rz = xr_z[0]
    cx = xc_x[0]
    cy = xc_y[0]
    cz = xc_z[0]
    dx = rx - cx
    dy = ry - cy
    dz = rz - cz
    work = jnp.sqrt(dx * dx + dy * dy + dz * dz + 1e-6)
    tn, n = work.shape
    iota_n = jax.lax.broadcasted_iota(jnp.int32, (tn, n), 1).astype(jnp.float32)
    lane_k = jax.lax.broadcasted_iota(jnp.int32, (tn, KPAD), 1)
    acc_idx = jnp.zeros((tn, KPAD), jnp.float32)
    acc_d = jnp.zeros((tn, KPAD), jnp.float32)
    acc_x = jnp.zeros((tn, KPAD), jnp.float32)
    acc_y = jnp.zeros((tn, KPAD), jnp.float32)
    acc_z = jnp.zeros((tn, KPAD), jnp.float32)
    for k in range(TOP_K):
        m = jnp.min(work, axis=1, keepdims=True)
        is_min = work == m
        idx = jnp.min(jnp.where(is_min, iota_n, float(n)), axis=1,
                      keepdims=True)
        chosen = iota_n == idx
        gx = jnp.sum(jnp.where(chosen, cx, 0.0), axis=1, keepdims=True)
        gy = jnp.sum(jnp.where(chosen, cy, 0.0), axis=1, keepdims=True)
        gz = jnp.sum(jnp.where(chosen, cz, 0.0), axis=1, keepdims=True)
        work = jnp.where(chosen, BIG, work)
        sel = lane_k == k
        acc_idx = jnp.where(sel, idx, acc_idx)
        acc_d = jnp.where(sel, m, acc_d)
        acc_x = jnp.where(sel, gx, acc_x)
        acc_y = jnp.where(sel, gy, acc_y)
        acc_z = jnp.where(sel, gz, acc_z)
    oidx[0] = acc_idx.astype(jnp.int32)
    od[0] = acc_d
    oxx[0] = acc_x
    oxy[0] = acc_y
    oxz[0] = acc_z


def _run_knn(xca, b, n, tn):
    col = lambda c: xca[:, :, c:c + 1]
    lane = lambda c: xca[:, :, c][:, None, :]
    row_spec = pl.BlockSpec((1, tn, 1), lambda i, t: (i, t, 0))
    lane_spec = pl.BlockSpec((1, 1, n), lambda i, t: (i, 0, 0))
    out_spec = pl.BlockSpec((1, tn, KPAD), lambda i, t: (i, t, 0))
    sd = jax.ShapeDtypeStruct
    outs = pl.pallas_call(
        _knn_body,
        grid=(b, n // tn),
        in_specs=[row_spec] * 3 + [lane_spec] * 3,
        out_specs=[out_spec] * 5,
        out_shape=[sd((b, n, KPAD), jnp.int32)] + [sd((b, n, KPAD), jnp.float32)] * 4,
    )(col(0), col(1), col(2), lane(0), lane(1), lane(2))
    return outs


def _edge_body(cols, whe, wsWe, wsbe, wve, ge, be, out):
    c = cols[...]
    idx_f = c[:, 0:1]
    n_i = c[:, 1:2]
    d = c[:, 2:3]
    dxx = c[:, 3:4] - c[:, 6:7]
    dxy = c[:, 4:5] - c[:, 7:8]
    dxz = c[:, 5:6] - c[:, 8:9]
    rt = c.shape[0]
    inv = 1.0 / jnp.sqrt(jnp.maximum(dxx * dxx + dxy * dxy + dxz * dxz, 1e-8))
    ux, uy, uz = dxx * inv, dxy * inv, dxz * inv
    mu = jax.lax.broadcasted_iota(jnp.int32, (rt, NUM_RBF), 1).astype(jnp.float32) * (20.0 / (NUM_RBF - 1))
    sig = 20.0 / NUM_RBF
    rbf = jnp.exp(-jnp.square((d - mu) / sig))
    j = jax.lax.broadcasted_iota(jnp.int32, (rt, NUM_POS // 2), 1).astype(jnp.float32)
    freq = jnp.exp(j * (-np.log(10000.0) / (NUM_POS // 2)))
    ang = (idx_f - n_i) * freq
    cosA = jnp.cos(ang)
    sinA = jnp.sin(ang)
    wh = whe[0:1, 0:1]
    wv = wve[0:1, 0:1]
    vhx, vhy, vhz = ux * wh, uy * wh, uz * wh
    vn = jnp.sqrt(jnp.maximum(vhx * vhx + vhy * vhy + vhz * vhz, 1e-8))
    s_cat = jnp.concatenate([rbf, cosA, sinA, vn], axis=1)
    s = jnp.dot(s_cat, wsWe[...], preferred_element_type=jnp.float32) + wsbe[...]
    mean = jnp.mean(s, axis=1, keepdims=True)
    var = jnp.mean(jnp.square(s - mean), axis=1, keepdims=True)
    s_ln = ge[...] * (s - mean) / jnp.sqrt(var + 1e-3) + be[...]
    out[...] = jnp.concatenate([vhx * wv, vhy * wv, vhz * wv, s_ln], axis=1)


def _run_edge(cols, whe, wsWe, wsbe, wve, ge, be, rows, rt):
    w_spec = lambda a: pl.BlockSpec(a.shape, lambda t: (0,) * a.ndim)
    out = pl.pallas_call(
        _edge_body,
        grid=(rows // rt,),
        in_specs=[pl.BlockSpec((rt, 9), lambda t: (t, 0))] +
                 [w_spec(a) for a in (whe, wsWe, wsbe, wve, ge, be)],
        out_specs=pl.BlockSpec((rt, 35), lambda t: (t, 0)),
        out_shape=jax.ShapeDtypeStruct((rows, 35), jnp.float32),
    )(cols, whe, wsWe, wsbe, wve, ge, be)
    return out


def _unit3(x, y, z):
    inv = 1.0 / jnp.sqrt(jnp.maximum(x * x + y * y + z * z, 1e-8))
    return x * inv, y * inv, z * inv


def _cross(ax, ay, az, bx, by, bz):
    return ay * bz - az * by, az * bx - ax * bz, ax * by - ay * bx


def _node_body(cols, whn, wsWn, wsbn, wvn, gn, bn, out):
    c = cols[...]
    v3 = lambda o: (c[:, o:o + 1], c[:, o + 1:o + 2], c[:, o + 2:o + 3])
    cam1, cm1, nn = v3(0), v3(3), v3(6)
    can, cn, np1, cap1 = v3(9), v3(12), v3(15), v3(18)
    m_first = c[:, 21:22]
    m_last = c[:, 22:23]
    sub = lambda a, b: (a[0] - b[0], a[1] - b[1], a[2] - b[2])
    u1 = _unit3(*sub(nn, cm1))
    u2 = _unit3(*sub(can, nn))
    u3 = _unit3(*sub(cn, can))
    u4 = _unit3(*sub(np1, cn))
    u5 = _unit3(*sub(cap1, np1))

    def dihed(a, bb, cc):
        n2 = _unit3(*_cross(*a, *bb))
        n1 = _unit3(*_cross(*bb, *cc))
        cosD = n2[0] * n1[0] + n2[1] * n1[1] + n2[2] * n1[2]
        cosD = jnp.clip(cosD, -1.0 + 1e-7, 1.0 - 1e-7)
        sgn = jnp.sign(a[0] * n1[0] + a[1] * n1[1] + a[2] * n1[2])
        return cosD, sgn * jnp.sqrt(1.0 - cosD * cosD)

    c0, s0 = dihed(u1, u2, u3)
    c1, s1 = dihed(u2, u3, u4)
    c2, s2 = dihed(u3, u4, u5)
    c0 = jnp.where(m_first > 0.5, 1.0, c0)
    s0 = jnp.where(m_first > 0.5, 0.0, s0)
    c1 = jnp.where(m_last > 0.5, 1.0, c1)
    s1 = jnp.where(m_last > 0.5, 0.0, s1)
    c2 = jnp.where(m_last > 0.5, 1.0, c2)
    s2 = jnp.where(m_last > 0.5, 0.0, s2)
    fw = _unit3(*sub(cap1, can))
    bw = _unit3(*sub(cam1, can))
    keep_l = 1.0 - m_last
    keep_f = 1.0 - m_first
    fw = (fw[0] * keep_l, fw[1] * keep_l, fw[2] * keep_l)
    bw = (bw[0] * keep_f, bw[1] * keep_f, bw[2] * keep_f)
    cv = _unit3(*sub(cn, can))
    nv = _unit3(*sub(nn, can))
    bis = _unit3(cv[0] + nv[0], cv[1] + nv[1], cv[2] + nv[2])
    perp = _unit3(*_cross(*cv, *nv))
    ca_, cb_ = -np.sqrt(1.0 / 3.0), -np.sqrt(2.0 / 3.0)
    vec = (bis[0] * ca_ + perp[0] * cb_,
           bis[1] * ca_ + perp[1] * cb_,
           bis[2] * ca_ + perp[2] * cb_)
    rt = c.shape[0]
    vh = []
    for sp in range(3):
        vh.append(vec[sp] * whn[0:1, :] + fw[sp] * whn[1:2, :] + bw[sp] * whn[2:3, :])
    vn = jnp.sqrt(jnp.maximum(vh[0] * vh[0] + vh[1] * vh[1] + vh[2] * vh[2], 1e-8))
    s_cat = jnp.concatenate([c0, c1, c2, s0, s1, s2, vn], axis=1)
    s = jnp.dot(s_cat, wsWn[...], preferred_element_type=jnp.float32) + wsbn[...]
    mean = jnp.mean(s, axis=1, keepdims=True)
    var = jnp.mean(jnp.square(s - mean), axis=1, keepdims=True)
    s_ln = gn[...] * (s - mean) / jnp.sqrt(var + 1e-3) + bn[...]
    vouts = [jnp.dot(vh[sp], wvn[...], preferred_element_type=jnp.float32)
             for sp in range(3)]
    out[...] = jnp.concatenate(vouts + [s_ln], axis=1)


def _run_node(cols, whn, wsWn, wsbn, wvn, gn, bn, rows, rt):
    w_spec = lambda a: pl.BlockSpec(a.shape, lambda t: (0,) * a.ndim)
    out = pl.pallas_call(
        _node_body,
        grid=(rows // rt,),
        in_specs=[pl.BlockSpec((rt, 23), lambda t: (t, 0))] +
                 [w_spec(a) for a in (whn, wsWn, wsbn, wvn, gn, bn)],
        out_specs=pl.BlockSpec((rt, 148), lambda t: (t, 0)),
        out_shape=jax.ShapeDtypeStruct((rows, 148), jnp.float32),
    )(cols, whn, wsWn, wsbn, wvn, gn, bn)
    return out


@jax.jit
def kernel(X, mask, wh_n, wsW_n, wsb_n, wv_n, wh_e, wsW_e, wsb_e, wv_e,
           gamma_n, beta_n, gamma_e, beta_e):
    b, n = X.shape[0], X.shape[1]
    xca = X[:, :, 1, :]
    idx32, d32, xx, xy, xz = _run_knn(xca, b, n, 256)
    E_idx = idx32[:, :, :TOP_K]

    rows_e = b * n * TOP_K
    flat = lambda a: a[:, :, :TOP_K].reshape(rows_e, 1)
    rep = lambda a: jnp.broadcast_to(a[:, :, None], (b, n, TOP_K)).reshape(rows_e, 1)
    n_i = jnp.broadcast_to(jnp.arange(n, dtype=jnp.float32)[None, :, None],
                           (b, n, TOP_K)).reshape(rows_e, 1)
    cols_e = jnp.concatenate([
        E_idx.astype(jnp.float32).reshape(rows_e, 1), n_i, flat(d32),
        flat(xx), flat(xy), flat(xz),
        rep(xca[:, :, 0]), rep(xca[:, :, 1]), rep(xca[:, :, 2]),
    ], axis=1)
    r2 = lambda a: a.reshape(1, -1)
    Eflat = _run_edge(cols_e, wh_e, wsW_e, r2(wsb_e), wv_e,
                      r2(gamma_e), r2(beta_e), rows_e, 2048)
    E_out = Eflat.reshape(b, n, TOP_K, 35)

    rows_n = b * n
    Na, CA, Cc = X[:, :, 0, :], X[:, :, 1, :], X[:, :, 2, :]
    prev = lambda a: jnp.pad(a[:, :-1, :], ((0, 0), (1, 0), (0, 0)))
    nxt = lambda a: jnp.pad(a[:, 1:, :], ((0, 0), (0, 1), (0, 0)))
    ar = jnp.arange(n)
    m_first = jnp.broadcast_to((ar == 0).astype(jnp.float32)[None, :, None], (b, n, 1))
    m_last = jnp.broadcast_to((ar == n - 1).astype(jnp.float32)[None, :, None], (b, n, 1))
    cols_n = jnp.concatenate([
        prev(CA), prev(Cc), Na, CA, Cc, nxt(Na), nxt(CA), m_first, m_last,
    ], axis=2).reshape(rows_n, 23)
    Vflat = _run_node(cols_n, wh_n, wsW_n, r2(wsb_n), wv_n,
                      r2(gamma_n), r2(beta_n), rows_n, 1024)
    V_out = Vflat.reshape(b, n, 148)
    return V_out, E_out, E_idx


# speedup vs baseline: 2.4048x; 1.0167x over previous
import functools
import numpy as np
import jax
import jax.numpy as jnp
from jax.experimental import pallas as pl

TOP_K = 30
KPAD = 32
NUM_RBF = 16
NUM_POS = 16
BIG = 1e30


def _knn_body(xr_x, xr_y, xr_z, xc_x, xc_y, xc_z, c4_ref,
              oidx, od, oxx, oxy, oxz):
    rx = xr_x[0]
    ry = xr_y[0]
    rz = xr_z[0]
    cx = xc_x[0]
    cy = xc_y[0]
    cz = xc_z[0]
    c4 = c4_ref[0]
    dx = rx - cx
    dy = ry - cy
    dz = rz - cz
    work = dx * dx + dy * dy + dz * dz
    tn, n = work.shape
    iota_n = jax.lax.broadcasted_iota(jnp.int32, (tn, n), 1).astype(jnp.float32)
    lane_k = jax.lax.broadcasted_iota(jnp.int32, (tn, KPAD), 1)
    acc_idx = jnp.zeros((tn, KPAD), jnp.float32)
    acc_d = jnp.zeros((tn, KPAD), jnp.float32)
    acc_x = jnp.zeros((tn, KPAD), jnp.float32)
    acc_y = jnp.zeros((tn, KPAD), jnp.float32)
    acc_z = jnp.zeros((tn, KPAD), jnp.float32)
    for k in range(TOP_K):
        m = jnp.min(work, axis=1, keepdims=True)
        idx = jnp.min(jnp.where(work == m, iota_n, float(n)), axis=1,
                      keepdims=True)
        chosen_f = (iota_n == idx).astype(jnp.float32)
        g = jnp.dot(chosen_f, c4, preferred_element_type=jnp.float32)
        work = jnp.where(chosen_f > 0.0, BIG, work)
        sel = lane_k == k
        acc_idx = jnp.where(sel, idx, acc_idx)
        acc_d = jnp.where(sel, jnp.sqrt(m + 1e-6), acc_d)
        acc_x = jnp.where(sel, g[:, 0:1], acc_x)
        acc_y = jnp.where(sel, g[:, 1:2], acc_y)
        acc_z = jnp.where(sel, g[:, 2:3], acc_z)
    oidx[0] = acc_idx.astype(jnp.int32)
    od[0] = acc_d
    oxx[0] = acc_x
    oxy[0] = acc_y
    oxz[0] = acc_z


def _run_knn(xca, b, n, tn):
    col = lambda c: xca[:, :, c:c + 1]
    lane = lambda c: xca[:, :, c][:, None, :]
    c4 = jnp.concatenate([xca, jnp.zeros((b, n, 1), jnp.float32)], axis=2)
    row_spec = pl.BlockSpec((1, tn, 1), lambda i, t: (i, t, 0))
    lane_spec = pl.BlockSpec((1, 1, n), lambda i, t: (i, 0, 0))
    c4_spec = pl.BlockSpec((1, n, 4), lambda i, t: (i, 0, 0))
    out_spec = pl.BlockSpec((1, tn, KPAD), lambda i, t: (i, t, 0))
    sd = jax.ShapeDtypeStruct
    outs = pl.pallas_call(
        _knn_body,
        grid=(b, n // tn),
        in_specs=[row_spec] * 3 + [lane_spec] * 3 + [c4_spec],
        out_specs=[out_spec] * 5,
        out_shape=[sd((b, n, KPAD), jnp.int32)] + [sd((b, n, KPAD), jnp.float32)] * 4,
    )(col(0), col(1), col(2), lane(0), lane(1), lane(2), c4)
    return outs


def _edge_body(cols, whe, wsWe, wsbe, wve, ge, be, out):
    c = cols[...]
    idx_f = c[:, 0:1]
    n_i = c[:, 1:2]
    d = c[:, 2:3]
    dxx = c[:, 3:4] - c[:, 6:7]
    dxy = c[:, 4:5] - c[:, 7:8]
    dxz = c[:, 5:6] - c[:, 8:9]
    rt = c.shape[0]
    inv = 1.0 / jnp.sqrt(jnp.maximum(dxx * dxx + dxy * dxy + dxz * dxz, 1e-8))
    ux, uy, uz = dxx * inv, dxy * inv, dxz * inv
    mu = jax.lax.broadcasted_iota(jnp.int32, (rt, NUM_RBF), 1).astype(jnp.float32) * (20.0 / (NUM_RBF - 1))
    sig = 20.0 / NUM_RBF
    rbf = jnp.exp(-jnp.square((d - mu) / sig))
    j = jax.lax.broadcasted_iota(jnp.int32, (rt, NUM_POS // 2), 1).astype(jnp.float32)
    freq = jnp.exp(j * (-np.log(10000.0) / (NUM_POS // 2)))
    ang = (idx_f - n_i) * freq
    cosA = jnp.cos(ang)
    sinA = jnp.sin(ang)
    wh = whe[0:1, 0:1]
    wv = wve[0:1, 0:1]
    vhx, vhy, vhz = ux * wh, uy * wh, uz * wh
    vn = jnp.sqrt(jnp.maximum(vhx * vhx + vhy * vhy + vhz * vhz, 1e-8))
    s_cat = jnp.concatenate([rbf, cosA, sinA, vn], axis=1)
    s = jnp.dot(s_cat, wsWe[...], preferred_element_type=jnp.float32) + wsbe[...]
    mean = jnp.mean(s, axis=1, keepdims=True)
    var = jnp.mean(jnp.square(s - mean), axis=1, keepdims=True)
    s_ln = ge[...] * (s - mean) / jnp.sqrt(var + 1e-3) + be[...]
    out[...] = jnp.concatenate([vhx * wv, vhy * wv, vhz * wv, s_ln], axis=1)


def _run_edge(cols, whe, wsWe, wsbe, wve, ge, be, rows, rt):
    w_spec = lambda a: pl.BlockSpec(a.shape, lambda t: (0,) * a.ndim)
    out = pl.pallas_call(
        _edge_body,
        grid=(rows // rt,),
        in_specs=[pl.BlockSpec((rt, 9), lambda t: (t, 0))] +
                 [w_spec(a) for a in (whe, wsWe, wsbe, wve, ge, be)],
        out_specs=pl.BlockSpec((rt, 35), lambda t: (t, 0)),
        out_shape=jax.ShapeDtypeStruct((rows, 35), jnp.float32),
    )(cols, whe, wsWe, wsbe, wve, ge, be)
    return out


def _unit3(x, y, z):
    inv = 1.0 / jnp.sqrt(jnp.maximum(x * x + y * y + z * z, 1e-8))
    return x * inv, y * inv, z * inv


def _cross(ax, ay, az, bx, by, bz):
    return ay * bz - az * by, az * bx - ax * bz, ax * by - ay * bx


def _node_body(cols, whn, wsWn, wsbn, wvn, gn, bn, out):
    c = cols[...]
    v3 = lambda o: (c[:, o:o + 1], c[:, o + 1:o + 2], c[:, o + 2:o + 3])
    cam1, cm1, nn = v3(0), v3(3), v3(6)
    can, cn, np1, cap1 = v3(9), v3(12), v3(15), v3(18)
    m_first = c[:, 21:22]
    m_last = c[:, 22:23]
    sub = lambda a, b: (a[0] - b[0], a[1] - b[1], a[2] - b[2])
    u1 = _unit3(*sub(nn, cm1))
    u2 = _unit3(*sub(can, nn))
    u3 = _unit3(*sub(cn, can))
    u4 = _unit3(*sub(np1, cn))
    u5 = _unit3(*sub(cap1, np1))

    def dihed(a, bb, cc):
        n2 = _unit3(*_cross(*a, *bb))
        n1 = _unit3(*_cross(*bb, *cc))
        cosD = n2[0] * n1[0] + n2[1] * n1[1] + n2[2] * n1[2]
        cosD = jnp.clip(cosD, -1.0 + 1e-7, 1.0 - 1e-7)
        sgn = jnp.sign(a[0] * n1[0] + a[1] * n1[1] + a[2] * n1[2])
        return cosD, sgn * jnp.sqrt(1.0 - cosD * cosD)

    c0, s0 = dihed(u1, u2, u3)
    c1, s1 = dihed(u2, u3, u4)
    c2, s2 = dihed(u3, u4, u5)
    c0 = jnp.where(m_first > 0.5, 1.0, c0)
    s0 = jnp.where(m_first > 0.5, 0.0, s0)
    c1 = jnp.where(m_last > 0.5, 1.0, c1)
    s1 = jnp.where(m_last > 0.5, 0.0, s1)
    c2 = jnp.where(m_last > 0.5, 1.0, c2)
    s2 = jnp.where(m_last > 0.5, 0.0, s2)
    fw = _unit3(*sub(cap1, can))
    bw = _unit3(*sub(cam1, can))
    keep_l = 1.0 - m_last
    keep_f = 1.0 - m_first
    fw = (fw[0] * keep_l, fw[1] * keep_l, fw[2] * keep_l)
    bw = (bw[0] * keep_f, bw[1] * keep_f, bw[2] * keep_f)
    cv = _unit3(*sub(cn, can))
    nv = _unit3(*sub(nn, can))
    bis = _unit3(cv[0] + nv[0], cv[1] + nv[1], cv[2] + nv[2])
    perp = _unit3(*_cross(*cv, *nv))
    ca_, cb_ = -np.sqrt(1.0 / 3.0), -np.sqrt(2.0 / 3.0)
    vec = (bis[0] * ca_ + perp[0] * cb_,
           bis[1] * ca_ + perp[1] * cb_,
           bis[2] * ca_ + perp[2] * cb_)
    rt = c.shape[0]
    vh = []
    for sp in range(3):
        vh.append(vec[sp] * whn[0:1, :] + fw[sp] * whn[1:2, :] + bw[sp] * whn[2:3, :])
    vn = jnp.sqrt(jnp.maximum(vh[0] * vh[0] + vh[1] * vh[1] + vh[2] * vh[2], 1e-8))
    s_cat = jnp.concatenate([c0, c1, c2, s0, s1, s2, vn], axis=1)
    s = jnp.dot(s_cat, wsWn[...], preferred_element_type=jnp.float32) + wsbn[...]
    mean = jnp.mean(s, axis=1, keepdims=True)
    var = jnp.mean(jnp.square(s - mean), axis=1, keepdims=True)
    s_ln = gn[...] * (s - mean) / jnp.sqrt(var + 1e-3) + bn[...]
    vouts = [jnp.dot(vh[sp], wvn[...], preferred_element_type=jnp.float32)
             for sp in range(3)]
    out[...] = jnp.concatenate(vouts + [s_ln], axis=1)


def _run_node(cols, whn, wsWn, wsbn, wvn, gn, bn, rows, rt):
    w_spec = lambda a: pl.BlockSpec(a.shape, lambda t: (0,) * a.ndim)
    out = pl.pallas_call(
        _node_body,
        grid=(rows // rt,),
        in_specs=[pl.BlockSpec((rt, 23), lambda t: (t, 0))] +
                 [w_spec(a) for a in (whn, wsWn, wsbn, wvn, gn, bn)],
        out_specs=pl.BlockSpec((rt, 148), lambda t: (t, 0)),
        out_shape=jax.ShapeDtypeStruct((rows, 148), jnp.float32),
    )(cols, whn, wsWn, wsbn, wvn, gn, bn)
    return out


@jax.jit
def kernel(X, mask, wh_n, wsW_n, wsb_n, wv_n, wh_e, wsW_e, wsb_e, wv_e,
           gamma_n, beta_n, gamma_e, beta_e):
    b, n = X.shape[0], X.shape[1]
    xca = X[:, :, 1, :]
    idx32, d32, xx, xy, xz = _run_knn(xca, b, n, 256)
    E_idx = idx32[:, :, :TOP_K]

    rows_e = b * n * TOP_K
    flat = lambda a: a[:, :, :TOP_K].reshape(rows_e, 1)
    rep = lambda a: jnp.broadcast_to(a[:, :, None], (b, n, TOP_K)).reshape(rows_e, 1)
    n_i = jnp.broadcast_to(jnp.arange(n, dtype=jnp.float32)[None, :, None],
                           (b, n, TOP_K)).reshape(rows_e, 1)
    cols_e = jnp.concatenate([
        E_idx.astype(jnp.float32).reshape(rows_e, 1), n_i, flat(d32),
        flat(xx), flat(xy), flat(xz),
        rep(xca[:, :, 0]), rep(xca[:, :, 1]), rep(xca[:, :, 2]),
    ], axis=1)
    r2 = lambda a: a.reshape(1, -1)
    Eflat = _run_edge(cols_e, wh_e, wsW_e, r2(wsb_e), wv_e,
                      r2(gamma_e), r2(beta_e), rows_e, 2048)
    E_out = Eflat.reshape(b, n, TOP_K, 35)

    rows_n = b * n
    Na, CA, Cc = X[:, :, 0, :], X[:, :, 1, :], X[:, :, 2, :]
    prev = lambda a: jnp.pad(a[:, :-1, :], ((0, 0), (1, 0), (0, 0)))
    nxt = lambda a: jnp.pad(a[:, 1:, :], ((0, 0), (0, 1), (0, 0)))
    ar = jnp.arange(n)
    m_first = jnp.broadcast_to((ar == 0).astype(jnp.float32)[None, :, None], (b, n, 1))
    m_last = jnp.broadcast_to((ar == n - 1).astype(jnp.float32)[None, :, None], (b, n, 1))
    cols_n = jnp.concatenate([
        prev(CA), prev(Cc), Na, CA, Cc, nxt(Na), nxt(CA), m_first, m_last,
    ], axis=2).reshape(rows_n, 23)
    Vflat = _run_node(cols_n, wh_n, wsW_n, r2(wsb_n), wv_n,
                      r2(gamma_n), r2(beta_n), rows_n, 1024)
    V_out = Vflat.reshape(b, n, 148)
    return V_out, E_out, E_idx
